# baseline (speedup 1.0000x reference)
import dataclasses
import functools

import jax
import jax.numpy as jnp
from jax import lax
from jax.experimental import pallas as pl
from jax.experimental.pallas import tpu as pltpu
from jax.experimental.pallas import tpu_sc as plsc

_N = 10000
_E = 320000
_NORM_SCALE = 1.0

_NW = 32
_EW = _E // _NW
_C = 125
_NCHUNK = _EW // _C
_G = 8
_NGROUP = _NCHUNK // _G
_NP = 10112
_ROWS_PER_SUB = _NP // 16
_DC = 128


def _spmm_sc(x_chunks, cols3, rows3, w2, zeros):
    K = len(x_chunks)
    mesh = plsc.VectorSubcoreMesh(core_axis_name="c", subcore_axis_name="s")

    def body(*refs):
        x_refs = refs[:K]
        cols_h, rows_h, w_h, zeros_h = refs[K:K + 4]
        out_refs = refs[K + 4:2 * K + 4]
        (cols_g, rows_g, w_g, gath_a, gath_b, acc_sh,
         sem_a, sem_b, sem_sa, sem_sb) = refs[2 * K + 4:]

        c = lax.axis_index("c")
        s = lax.axis_index("s")
        wid = s * 2 + c

        def start_gather(xk, t, gath, sem):
            pltpu.async_copy(xk.at[cols_g.at[t]], gath, sem)

        def wait_gather(xk, t, gath, sem):
            pltpu.make_async_copy(xk.at[cols_g.at[t]], gath, sem).wait()

        def scale(xk, t, gath, sem):
            wait_gather(xk, t, gath, sem)
            t16 = lax.broadcast_in_dim(t, (16,), ())

            @plsc.parallel_loop(0, _C, 1, unroll=5)
            def _edge(e):
                e16 = lax.broadcast_in_dim(e, (16,), ())
                wv = plsc.load_gather(w_g, [t16, e16])
                for j in range(_DC // 16):
                    sl = pl.ds(j * 16, 16)
                    gath[e, sl] = gath[e, sl] * wv

        def start_scatter(t, gath, sem):
            pltpu.async_copy(gath, acc_sh.at[rows_g.at[t]], sem, add=True)

        def wait_scatter(t, gath, sem):
            pltpu.make_async_copy(gath, acc_sh.at[rows_g.at[t]], sem).wait()

        for k in range(K):
            xk = x_refs[k]
            outk = out_refs[k]

            pltpu.sync_copy(zeros_h, acc_sh.at[pl.ds(s * _ROWS_PER_SUB,
                                                     _ROWS_PER_SUB)])
            plsc.subcore_barrier()

            @pl.loop(0, _NGROUP)
            def _group(g):
                pltpu.sync_copy(cols_h.at[wid, g], cols_g)
                pltpu.sync_copy(rows_h.at[wid, g], rows_g)
                pltpu.sync_copy(w_h.at[wid, g], w_g)

                start_gather(xk, 0, gath_a, sem_a)

                @pl.loop(0, _G // 2)
                def _pair(p):
                    @pl.when(p > 0)
                    def _():
                        wait_scatter(2 * p - 1, gath_b, sem_sb)
                    start_gather(xk, 2 * p + 1, gath_b, sem_b)
                    scale(xk, 2 * p, gath_a, sem_a)
                    start_scatter(2 * p, gath_a, sem_sa)
                    wait_scatter(2 * p, gath_a, sem_sa)

                    @pl.when(p < _G // 2 - 1)
                    def _():
                        start_gather(xk, 2 * p + 2, gath_a, sem_a)
                    scale(xk, 2 * p + 1, gath_b, sem_b)
                    start_scatter(2 * p + 1, gath_b, sem_sb)

                wait_scatter(_G - 1, gath_b, sem_sb)

            plsc.subcore_barrier()
            pltpu.sync_copy(
                acc_sh.at[pl.ds(s * _ROWS_PER_SUB, _ROWS_PER_SUB)],
                outk.at[pl.ds(c * _NP + s * _ROWS_PER_SUB, _ROWS_PER_SUB)])

    out_types = [jax.ShapeDtypeStruct((2 * _NP, _DC), jnp.float32)
                 for _ in range(K)]
    cp = pltpu.CompilerParams()
    if "needs_layout_passes" in pltpu.CompilerParams.__dataclass_fields__:
        cp = dataclasses.replace(cp, needs_layout_passes=False)
    kern = pl.kernel(
        body,
        out_type=out_types,
        mesh=mesh,
        compiler_params=cp,
        scratch_types=[
            pltpu.VMEM((_G, _C), jnp.int32),
            pltpu.VMEM((_G, _C), jnp.int32),
            pltpu.VMEM((_G, _C), jnp.float32),
            pltpu.VMEM((_C, _DC), jnp.float32),
            pltpu.VMEM((_C, _DC), jnp.float32),
            pltpu.VMEM_SHARED((_NP, _DC), jnp.float32),
            pltpu.SemaphoreType.DMA,
            pltpu.SemaphoreType.DMA,
            pltpu.SemaphoreType.DMA,
            pltpu.SemaphoreType.DMA,
        ],
    )
    out = kern(*x_chunks, cols3, rows3, w2, zeros)
    return list(out) if isinstance(out, (tuple, list)) else [out]


def _dense_body(*refs, k_in, k_out, apply_pn):
    in_refs = refs[:k_in]
    w_ref, b_ref = refs[k_in:k_in + 2]
    out_refs = refs[k_in + 2:]
    y = b_ref[...]
    for k in range(k_in):
        hk = in_refs[k][:_N, :] + in_refs[k][_NP:_NP + _N, :]
        y = y + jnp.dot(hk, w_ref[pl.ds(k * _DC, _DC), :],
                        preferred_element_type=jnp.float32)
    if apply_pn:
        yc = y - jnp.mean(y, axis=0, keepdims=True)
        rn = jnp.sqrt(1e-6 + jnp.mean(jnp.sum(yc * yc, axis=1)))
        y = jnp.maximum((_NORM_SCALE / rn) * yc, 0.0)
    for k in range(k_out):
        out_refs[k][...] = y[:, k * _DC:(k + 1) * _DC]


def _dense_tc(partials, W, b, apply_pn, k_out):
    k_in = len(partials)
    out_shape = [jax.ShapeDtypeStruct((_N, _DC), jnp.float32)
                 for _ in range(k_out)]
    out = pl.pallas_call(
        functools.partial(_dense_body, k_in=k_in, k_out=k_out,
                          apply_pn=apply_pn),
        out_shape=out_shape,
    )(*partials, W, b.reshape(1, -1))
    return list(out)


def kernel(x, edge_index, edge_weight, W0, b0, W1, b1, Wout, bout):
    cols3 = edge_index[1].reshape(_NW, _NGROUP, _G, _C)
    rows3 = edge_index[0].reshape(_NW, _NGROUP, _G, _C)
    w2 = edge_weight.reshape(_NW, _NGROUP, _G, _C)
    zeros = jnp.zeros((_ROWS_PER_SUB, _DC), jnp.float32)

    h = [x]
    p = _spmm_sc(h, cols3, rows3, w2, zeros)
    h = _dense_tc(p, W0, b0, True, 2)
    p = _spmm_sc(h, cols3, rows3, w2, zeros)
    h = _dense_tc(p, W1, b1, True, 2)
    p = _spmm_sc(h, cols3, rows3, w2, zeros)
    out = _dense_tc(p, Wout, bout, False, 1)
    return out[0]


# speedup vs baseline: 1.1877x; 1.1877x over previous
import dataclasses
import functools

import jax
import jax.numpy as jnp
from jax import lax
from jax.experimental import pallas as pl
from jax.experimental.pallas import tpu as pltpu
from jax.experimental.pallas import tpu_sc as plsc

_N = 10000
_E = 320000
_NORM_SCALE = 1.0

_NW = 32
_EW = _E // _NW
_C = 80
_NCHUNK = _EW // _C
_G = 25
_NGROUP = _NCHUNK // _G
_NP = 10112
_ROWS_PER_SUB = _NP // 16
_DC = 128


def _spmm_sc(x_chunks, cols3, rows3, w2, zeros):
    K = len(x_chunks)
    mesh = plsc.VectorSubcoreMesh(core_axis_name="c", subcore_axis_name="s")

    def body(*refs):
        x_refs = refs[:K]
        cols_h, rows_h, w_h, zeros_h = refs[K:K + 4]
        out_refs = refs[K + 4:2 * K + 4]
        (cols_g, rows_g, w_g, gath0, gath1, gath2, acc_sh,
         sem0, sem1, sem2, sem_s0, sem_s1, sem_s2) = refs[2 * K + 4:]
        bufs = (gath0, gath1, gath2)
        sems = (sem0, sem1, sem2)
        sem_s = (sem_s0, sem_s1, sem_s2)

        c = lax.axis_index("c")
        s = lax.axis_index("s")
        wid = s * 2 + c

        def start_gather(xk, t, gath, sem):
            pltpu.async_copy(xk.at[cols_g.at[t]], gath, sem)

        def wait_gather(xk, t, gath, sem):
            pltpu.make_async_copy(xk.at[cols_g.at[t]], gath, sem).wait()

        def scale(xk, t, gath, sem):
            wait_gather(xk, t, gath, sem)
            t16 = lax.broadcast_in_dim(t, (16,), ())

            @plsc.parallel_loop(0, _C, 1, unroll=4)
            def _edge(e):
                e16 = lax.broadcast_in_dim(e, (16,), ())
                wv = plsc.load_gather(w_g, [t16, e16])
                for j in range(_DC // 16):
                    sl = pl.ds(j * 16, 16)
                    gath[e, sl] = gath[e, sl] * wv

        def start_scatter(t, gath, sem):
            pltpu.async_copy(gath, acc_sh.at[rows_g.at[t]], sem, add=True)

        def wait_scatter(t, gath, sem):
            pltpu.make_async_copy(gath, acc_sh.at[rows_g.at[t]], sem).wait()

        for k in range(K):
            xk = x_refs[k]
            outk = out_refs[k]

            pltpu.sync_copy(zeros_h, acc_sh.at[pl.ds(s * _ROWS_PER_SUB,
                                                     _ROWS_PER_SUB)])
            plsc.subcore_barrier()

            @pl.loop(0, _NGROUP)
            def _group(g):
                pltpu.sync_copy(cols_h.at[wid, g], cols_g)
                pltpu.sync_copy(rows_h.at[wid, g], rows_g)
                pltpu.sync_copy(w_h.at[wid, g], w_g)

                start_gather(xk, 0, bufs[0], sems[0])
                start_gather(xk, 1, bufs[1], sems[1])

                @pl.loop(0, (_G - 1) // 3)
                def _tri(i):
                    for b in range(3):
                        t = 3 * i + b
                        scale(xk, t, bufs[b], sems[b])
                        start_scatter(t, bufs[b], sem_s[b])
                        if b == 0:
                            @pl.when(i > 0)
                            def _():
                                wait_scatter(3 * i - 1, bufs[2], sem_s[2])
                        else:
                            wait_scatter(t - 1, bufs[b - 1], sem_s[b - 1])
                        if b == 2:
                            @pl.when(i < (_G - 1) // 3 - 1)
                            def _():
                                start_gather(xk, t + 2, bufs[1], sems[1])
                        else:
                            nb = (b + 2) % 3
                            start_gather(xk, t + 2, bufs[nb], sems[nb])

                scale(xk, _G - 1, bufs[0], sems[0])
                wait_scatter(_G - 2, bufs[2], sem_s[2])
                pltpu.sync_copy(bufs[0], acc_sh.at[rows_g.at[_G - 1]],
                                add=True)

            plsc.subcore_barrier()
            pltpu.sync_copy(
                acc_sh.at[pl.ds(s * _ROWS_PER_SUB, _ROWS_PER_SUB)],
                outk.at[pl.ds(c * _NP + s * _ROWS_PER_SUB, _ROWS_PER_SUB)])

    out_types = [jax.ShapeDtypeStruct((2 * _NP, _DC), jnp.float32)
                 for _ in range(K)]
    cp = pltpu.CompilerParams()
    if "needs_layout_passes" in pltpu.CompilerParams.__dataclass_fields__:
        cp = dataclasses.replace(cp, needs_layout_passes=False)
    kern = pl.kernel(
        body,
        out_type=out_types,
        mesh=mesh,
        compiler_params=cp,
        scratch_types=[
            pltpu.VMEM((_G, _C), jnp.int32),
            pltpu.VMEM((_G, _C), jnp.int32),
            pltpu.VMEM((_G, _C), jnp.float32),
            pltpu.VMEM((_C, _DC), jnp.float32),
            pltpu.VMEM((_C, _DC), jnp.float32),
            pltpu.VMEM((_C, _DC), jnp.float32),
            pltpu.VMEM_SHARED((_NP, _DC), jnp.float32),
            pltpu.SemaphoreType.DMA,
            pltpu.SemaphoreType.DMA,
            pltpu.SemaphoreType.DMA,
            pltpu.SemaphoreType.DMA,
            pltpu.SemaphoreType.DMA,
            pltpu.SemaphoreType.DMA,
        ],
    )
    out = kern(*x_chunks, cols3, rows3, w2, zeros)
    return list(out) if isinstance(out, (tuple, list)) else [out]


def _dense_body(*refs, k_in, k_out, apply_pn):
    in_refs = refs[:k_in]
    w_ref, b_ref = refs[k_in:k_in + 2]
    out_refs = refs[k_in + 2:]
    y = b_ref[...]
    for k in range(k_in):
        hk = in_refs[k][:_N, :] + in_refs[k][_NP:_NP + _N, :]
        y = y + jnp.dot(hk, w_ref[pl.ds(k * _DC, _DC), :],
                        preferred_element_type=jnp.float32)
    if apply_pn:
        yc = y - jnp.mean(y, axis=0, keepdims=True)
        rn = jnp.sqrt(1e-6 + jnp.mean(jnp.sum(yc * yc, axis=1)))
        y = jnp.maximum((_NORM_SCALE / rn) * yc, 0.0)
    for k in range(k_out):
        out_refs[k][...] = y[:, k * _DC:(k + 1) * _DC]


def _dense_tc(partials, W, b, apply_pn, k_out):
    k_in = len(partials)
    out_shape = [jax.ShapeDtypeStruct((_N, _DC), jnp.float32)
                 for _ in range(k_out)]
    out = pl.pallas_call(
        functools.partial(_dense_body, k_in=k_in, k_out=k_out,
                          apply_pn=apply_pn),
        out_shape=out_shape,
    )(*partials, W, b.reshape(1, -1))
    return list(out)


def kernel(x, edge_index, edge_weight, W0, b0, W1, b1, Wout, bout):
    cols3 = edge_index[1].reshape(_NW, _NGROUP, _G, _C)
    rows3 = edge_index[0].reshape(_NW, _NGROUP, _G, _C)
    w2 = edge_weight.reshape(_NW, _NGROUP, _G, _C)
    zeros = jnp.zeros((_ROWS_PER_SUB, _DC), jnp.float32)

    h = [x]
    p = _spmm_sc(h, cols3, rows3, w2, zeros)
    h = _dense_tc(p, W0, b0, True, 2)
    p = _spmm_sc(h, cols3, rows3, w2, zeros)
    h = _dense_tc(p, W1, b1, True, 2)
    p = _spmm_sc(h, cols3, rows3, w2, zeros)
    out = _dense_tc(p, Wout, bout, False, 1)
    return out[0]
